# baseline (speedup 1.0000x reference)
import jax
import jax.numpy as jnp
from jax.experimental import pallas as pl
from jax.experimental.pallas import tpu as pltpu
from jax.experimental.pallas import tpu_sc as plsc

_WINDOW = 128
_NW = 32


def _embed_sc(tokens_flat, table):
    n_idx = tokens_flat.shape[0]
    emb = table.shape[1]
    mesh = plsc.VectorSubcoreMesh(core_axis_name="core",
                                  subcore_axis_name="subcore")

    @pl.kernel(
        out_type=jax.ShapeDtypeStruct((n_idx, emb), table.dtype),
        mesh=mesh,
        compiler_params=pltpu.CompilerParams(use_tc_tiling_on_sc=False),
    )
    def k(table_hbm, idx_hbm, out_hbm):
        def body(idx_vmem, out_vmem):
            pltpu.sync_copy(table_hbm.at[idx_vmem.at[0]], out_vmem)

        pltpu.emit_pipeline(
            body,
            grid=(n_idx // _WINDOW,),
            in_specs=[pl.BlockSpec((1, _WINDOW), index_map=lambda i: (0, i))],
            out_specs=[pl.BlockSpec((_WINDOW, emb), index_map=lambda i: (i, 0))],
            core_axis_name=("core", "subcore"),
            dimension_semantics=(pltpu.PARALLEL,),
        )(idx_hbm, out_hbm)

    return k(table, tokens_flat.reshape(1, n_idx))


_CH = 128


def _repad_sc(packed, n_idx, emb):
    rows = packed.shape[0]
    rows_per_w = rows // _NW
    chunks = rows_per_w // _CH

    mesh = plsc.VectorSubcoreMesh(core_axis_name="core",
                                  subcore_axis_name="subcore")

    @pl.kernel(
        out_type=jax.ShapeDtypeStruct((n_idx, emb), jnp.float32),
        mesh=mesh,
        scratch_types=[
            pltpu.VMEM((_CH, 128), jnp.float32),
            pltpu.VMEM((_CH, 128), jnp.float32),
            pltpu.VMEM((2 * _CH, emb), jnp.float32),
            pltpu.SemaphoreType.DMA,
            pltpu.SemaphoreType.DMA,
        ],
    )
    def k(in_hbm, out_hbm, b0, b1, ubuf, s0, s1):
        wid = jax.lax.axis_index("subcore") * 2 + jax.lax.axis_index("core")
        base = wid * rows_per_w

        def fetch(c, buf, sem):
            return pltpu.async_copy(
                in_hbm.at[pl.ds(base + c * _CH, _CH)], buf, sem)

        def expand(c, buf):
            @pl.loop(0, _CH)
            def _(j):
                for h in range(2):
                    for q in range(0, emb, 16):
                        ubuf[2 * j + h, pl.ds(q, 16)] = (
                            buf[j, pl.ds(h * emb + q, 16)])
            pltpu.sync_copy(
                ubuf, out_hbm.at[pl.ds(2 * (base + c * _CH), 2 * _CH)])

        fetch(0, b0, s0).wait()

        @pl.loop(0, chunks - 1)
        def _(c):
            even = c % 2 == 0

            @pl.when(even)
            def _():
                cp = fetch(c + 1, b1, s1)
                expand(c, b0)
                cp.wait()

            @pl.when(jnp.logical_not(even))
            def _():
                cp = fetch(c + 1, b0, s0)
                expand(c, b1)
                cp.wait()

        @pl.when((chunks - 1) % 2 == 0)
        def _():
            expand(chunks - 1, b0)

        @pl.when((chunks - 1) % 2 == 1)
        def _():
            expand(chunks - 1, b1)

    return k(packed)


def kernel(tokens, table):
    batch, hist = tokens.shape
    n_idx = batch * hist
    emb = table.shape[1]
    flat = tokens.reshape(n_idx)
    out = _embed_sc(flat, table)
    out = _repad_sc(out.reshape(n_idx // 2, 2 * emb), n_idx, emb)
    return out.reshape(batch, hist, emb)
